# Initial kernel scaffold; baseline (speedup 1.0000x reference)
import functools

import jax
import jax.numpy as jnp
from jax import lax
from jax.experimental import pallas as pl
from jax.experimental.pallas import tpu as pltpu
from jax.experimental.pallas import tpu_sc as plsc

N = 10000
E = 320000
D = 128
EPS = 1e-5
NP = N + 8

NC = 2
NS = 16
NW = NC * NS
EPW = E // NW
CH = 80
NCH = EPW // CH
RPW = N // NS
ZCH = 125

_mesh = plsc.VectorSubcoreMesh(
    core_axis_name="c", subcore_axis_name="s", num_cores=NC, num_subcores=NS
)


@functools.partial(
    pl.kernel,
    mesh=_mesh,
    out_type=jax.ShapeDtypeStruct((NW, N), jnp.float32),
    scratch_types=[
        pltpu.VMEM((EPW,), jnp.int32),
        pltpu.VMEM((EPW,), jnp.int32),
        pltpu.VMEM((N,), jnp.float32),
    ],
)
def _deg_kernel(src_hbm, dst_hbm, out_hbm, src_v, dst_v, acc_v):
    cid = lax.axis_index("c")
    sid = lax.axis_index("s")
    wid = sid * NC + cid
    base = wid * EPW
    pltpu.sync_copy(src_hbm.at[pl.ds(base, EPW)], src_v)
    pltpu.sync_copy(dst_hbm.at[pl.ds(base, EPW)], dst_v)

    zeros16 = jnp.zeros((16,), jnp.float32)

    def zero_body(i, carry):
        acc_v[pl.ds(i * 16, 16)] = zeros16
        return carry

    lax.fori_loop(0, N // 16, zero_body, 0)

    def edge_body(i, carry):
        s = src_v[pl.ds(i * 16, 16)]
        d = dst_v[pl.ds(i * 16, 16)]
        w = jnp.where(s != d, 1.0, 0.0).astype(jnp.float32)
        plsc.addupdate_scatter(acc_v, [s], w)
        return carry

    lax.fori_loop(0, EPW // 16, edge_body, 0)
    pltpu.sync_copy(acc_v, out_hbm.at[wid])


@functools.partial(
    pl.kernel,
    mesh=_mesh,
    out_type=jax.ShapeDtypeStruct((NC, N, D), jnp.float32),
    scratch_types=[
        pltpu.VMEM((CH,), jnp.int32),
        pltpu.VMEM((CH,), jnp.int32),
        pltpu.VMEM((CH, D), jnp.float32),
        pltpu.VMEM((ZCH, D), jnp.float32),
        pltpu.VMEM_SHARED((N, D), jnp.float32),
        pltpu.SemaphoreType.DMA,
    ],
)
def _prop_kernel(u_hbm, srcp_hbm, dst_hbm, out_hbm,
                 sidx_v, didx_v, rows_v, zero_v, acc_sh, gsem):
    cid = lax.axis_index("c")
    sid = lax.axis_index("s")
    wid = sid * NC + cid

    zeros16 = jnp.zeros((16,), jnp.float32)

    def zbuf_body(i, carry):
        r = i // (D // 16)
        c = i % (D // 16)
        zero_v[r, pl.ds(c * 16, 16)] = zeros16
        return carry

    lax.fori_loop(0, ZCH * (D // 16), zbuf_body, 0)

    for j in range(RPW // ZCH):
        pltpu.sync_copy(zero_v, acc_sh.at[pl.ds(sid * RPW + j * ZCH, ZCH)])
    plsc.subcore_barrier()

    def chunk_body(g, carry):
        base = wid * EPW + g * CH
        pltpu.sync_copy(srcp_hbm.at[pl.ds(base, CH)], sidx_v)
        pltpu.sync_copy(dst_hbm.at[pl.ds(base, CH)], didx_v)
        pltpu.async_copy(u_hbm.at[sidx_v], rows_v, gsem).wait()
        pltpu.sync_copy(rows_v, acc_sh.at[didx_v], add=True)
        return carry

    lax.fori_loop(0, NCH, chunk_body, 0)
    plsc.subcore_barrier()

    for j in range(RPW // ZCH):
        off = sid * RPW + j * ZCH
        pltpu.sync_copy(acc_sh.at[pl.ds(off, ZCH)],
                        out_hbm.at[cid, pl.ds(off, ZCH)])


def _tcA_body(degp_ref, x_ref, u0_ref, dinv_ref):
    deg = jnp.sum(degp_ref[...], axis=0)
    dinv = jnp.where(deg > 0.0, lax.rsqrt(jnp.maximum(deg, 1.0)), 0.0)
    dv = dinv[:, None]
    dinv_ref[...] = dv
    u0_ref[pl.ds(0, N), :] = x_ref[...] * dv
    u0_ref[pl.ds(N, NP - N), :] = jnp.zeros((NP - N, D), jnp.float32)


def _tcB_body(s1_ref, dinv_ref, tx1_ref, u1_ref):
    s = s1_ref[0] + s1_ref[1]
    dv = dinv_ref[...]
    tx1 = -(dv * s)
    tx1_ref[...] = tx1
    u1_ref[pl.ds(0, N), :] = dv * tx1
    u1_ref[pl.ds(N, NP - N), :] = jnp.zeros((NP - N, D), jnp.float32)


def _tcC_body(x_ref, tx1_ref, s2_ref, dinv_ref, w_ref, b_ref, g_ref, be_ref,
              y_ref):
    x = x_ref[...]
    tx1 = tx1_ref[...]
    dv = dinv_ref[...]
    tx2 = -2.0 * (dv * (s2_ref[0] + s2_ref[1])) - x
    out = jnp.dot(x, w_ref[0], preferred_element_type=jnp.float32)
    out += jnp.dot(tx1, w_ref[1], preferred_element_type=jnp.float32)
    out += jnp.dot(tx2, w_ref[2], preferred_element_type=jnp.float32)
    out += b_ref[...]
    mean = jnp.mean(out, axis=0, keepdims=True)
    var = jnp.mean((out - mean) ** 2, axis=0, keepdims=True)
    y_ref[...] = (out - mean) * lax.rsqrt(var + EPS) * g_ref[...] + be_ref[...]


_tcA = pl.pallas_call(
    _tcA_body,
    out_shape=(
        jax.ShapeDtypeStruct((NP, D), jnp.float32),
        jax.ShapeDtypeStruct((N, 1), jnp.float32),
    ),
)

_tcB = pl.pallas_call(
    _tcB_body,
    out_shape=(
        jax.ShapeDtypeStruct((N, D), jnp.float32),
        jax.ShapeDtypeStruct((NP, D), jnp.float32),
    ),
)

_tcC = pl.pallas_call(
    _tcC_body,
    out_shape=jax.ShapeDtypeStruct((N, D), jnp.float32),
)


def kernel(x, edge_index, W, b, gamma, beta):
    src = edge_index[0]
    dst = edge_index[1]
    srcp = jnp.where(src == dst, N, src)

    degp = _deg_kernel(src, dst)
    u0, dinv = _tcA(degp, x)
    s1 = _prop_kernel(u0, srcp, dst)
    tx1, u1 = _tcB(s1, dinv)
    s2 = _prop_kernel(u1, srcp, dst)
    return _tcC(x, tx1, s2, dinv,
                W, b.reshape(1, D), gamma.reshape(1, D), beta.reshape(1, D))


# speedup vs baseline: 10.8657x; 10.8657x over previous
import functools

import jax
import jax.numpy as jnp
from jax import lax
from jax.experimental import pallas as pl
from jax.experimental.pallas import tpu as pltpu
from jax.experimental.pallas import tpu_sc as plsc

N = 10000
E = 320000
D = 128
EPS = 1e-5
NP = N + 8

NC = 2
NS = 16
NW = NC * NS
EPW = E // NW
CH = 80
NCH = EPW // CH
ACCN = 10240
RPW = ACCN // NS
ZCH = 128

_mesh = plsc.VectorSubcoreMesh(
    core_axis_name="c", subcore_axis_name="s", num_cores=NC, num_subcores=NS
)

_sc_params = pltpu.CompilerParams(needs_layout_passes=False)


@functools.partial(
    pl.kernel,
    mesh=_mesh,
    out_type=jax.ShapeDtypeStruct((NW, N), jnp.float32),
    scratch_types=[
        pltpu.VMEM((EPW,), jnp.int32),
        pltpu.VMEM((EPW,), jnp.int32),
        pltpu.VMEM((N,), jnp.float32),
    ],
    compiler_params=_sc_params,
)
def _deg_kernel(src_hbm, dst_hbm, out_hbm, src_v, dst_v, acc_v):
    cid = lax.axis_index("c")
    sid = lax.axis_index("s")
    wid = sid * NC + cid
    base = wid * EPW
    pltpu.sync_copy(src_hbm.at[pl.ds(base, EPW)], src_v)
    pltpu.sync_copy(dst_hbm.at[pl.ds(base, EPW)], dst_v)

    zeros16 = jnp.zeros((16,), jnp.float32)

    def zero_body(i, carry):
        acc_v[pl.ds(i * 16, 16)] = zeros16
        return carry

    lax.fori_loop(0, N // 16, zero_body, 0)

    def edge_body(i, carry):
        s = src_v[pl.ds(i * 16, 16)]
        d = dst_v[pl.ds(i * 16, 16)]
        w = jnp.where(s != d, 1.0, 0.0).astype(jnp.float32)
        plsc.addupdate_scatter(acc_v, [s], w)
        return carry

    lax.fori_loop(0, EPW // 16, edge_body, 0)
    pltpu.sync_copy(acc_v, out_hbm.at[wid])


@functools.partial(
    pl.kernel,
    mesh=_mesh,
    out_type=jax.ShapeDtypeStruct((NC, ACCN, D), jnp.float32),
    scratch_types=[
        pltpu.VMEM((CH,), jnp.int32),
        pltpu.VMEM((CH,), jnp.int32),
        pltpu.VMEM((CH, D), jnp.float32),
        pltpu.VMEM((ZCH, D), jnp.float32),
        pltpu.VMEM_SHARED((ACCN, D), jnp.float32),
        pltpu.SemaphoreType.DMA,
    ],
    compiler_params=_sc_params,
)
def _prop_kernel(u_hbm, srcp_hbm, dst_hbm, out_hbm,
                 sidx_v, didx_v, rows_v, zero_v, acc_sh, gsem):
    cid = lax.axis_index("c")
    sid = lax.axis_index("s")
    wid = sid * NC + cid

    zeros16 = jnp.zeros((16,), jnp.float32)

    def zbuf_body(i, carry):
        r = i // (D // 16)
        c = i % (D // 16)
        zero_v[r, pl.ds(c * 16, 16)] = zeros16
        return carry

    lax.fori_loop(0, ZCH * (D // 16), zbuf_body, 0)

    for j in range(RPW // ZCH):
        pltpu.sync_copy(zero_v, acc_sh.at[pl.ds(sid * RPW + j * ZCH, ZCH)])
    plsc.subcore_barrier()

    def chunk_body(g, carry):
        base = wid * EPW + g * CH
        pltpu.sync_copy(srcp_hbm.at[pl.ds(base, CH)], sidx_v)
        pltpu.sync_copy(dst_hbm.at[pl.ds(base, CH)], didx_v)
        pltpu.async_copy(u_hbm.at[sidx_v], rows_v, gsem).wait()
        pltpu.sync_copy(rows_v, acc_sh.at[didx_v], add=True)
        return carry

    lax.fori_loop(0, NCH, chunk_body, 0)
    plsc.subcore_barrier()

    off = sid * RPW
    pltpu.sync_copy(acc_sh.at[pl.ds(off, RPW)],
                    out_hbm.at[cid, pl.ds(off, RPW)])


def _tcA_body(degp_ref, x_ref, u0_ref, dinv_ref):
    deg = jnp.sum(degp_ref[...], axis=0)
    dinv = jnp.where(deg > 0.0, lax.rsqrt(jnp.maximum(deg, 1.0)), 0.0)
    dv = dinv[:, None]
    dinv_ref[...] = dv
    u0_ref[pl.ds(0, N), :] = x_ref[...] * dv
    u0_ref[pl.ds(N, NP - N), :] = jnp.zeros((NP - N, D), jnp.float32)


def _tcB_body(s1_ref, dinv_ref, tx1_ref, u1_ref):
    s = s1_ref[0, pl.ds(0, N), :] + s1_ref[1, pl.ds(0, N), :]
    dv = dinv_ref[...]
    tx1 = -(dv * s)
    tx1_ref[...] = tx1
    u1_ref[pl.ds(0, N), :] = dv * tx1
    u1_ref[pl.ds(N, NP - N), :] = jnp.zeros((NP - N, D), jnp.float32)


def _tcC_body(x_ref, tx1_ref, s2_ref, dinv_ref, w_ref, b_ref, g_ref, be_ref,
              y_ref):
    x = x_ref[...]
    tx1 = tx1_ref[...]
    dv = dinv_ref[...]
    s2 = s2_ref[0, pl.ds(0, N), :] + s2_ref[1, pl.ds(0, N), :]
    tx2 = -2.0 * (dv * s2) - x
    out = jnp.dot(x, w_ref[0], preferred_element_type=jnp.float32)
    out += jnp.dot(tx1, w_ref[1], preferred_element_type=jnp.float32)
    out += jnp.dot(tx2, w_ref[2], preferred_element_type=jnp.float32)
    out += b_ref[...]
    mean = jnp.mean(out, axis=0, keepdims=True)
    var = jnp.mean((out - mean) ** 2, axis=0, keepdims=True)
    y_ref[...] = (out - mean) * lax.rsqrt(var + EPS) * g_ref[...] + be_ref[...]


_tcA = pl.pallas_call(
    _tcA_body,
    out_shape=(
        jax.ShapeDtypeStruct((NP, D), jnp.float32),
        jax.ShapeDtypeStruct((N, 1), jnp.float32),
    ),
)

_tcB = pl.pallas_call(
    _tcB_body,
    out_shape=(
        jax.ShapeDtypeStruct((N, D), jnp.float32),
        jax.ShapeDtypeStruct((NP, D), jnp.float32),
    ),
)

_tcC = pl.pallas_call(
    _tcC_body,
    out_shape=jax.ShapeDtypeStruct((N, D), jnp.float32),
)


def kernel(x, edge_index, W, b, gamma, beta):
    src = edge_index[0]
    dst = edge_index[1]
    srcp = jnp.where(src == dst, N, src)

    degp = _deg_kernel(src, dst)
    u0, dinv = _tcA(degp, x)
    s1 = _prop_kernel(u0, srcp, dst)
    tx1, u1 = _tcB(s1, dinv)
    s2 = _prop_kernel(u1, srcp, dst)
    return _tcC(x, tx1, s2, dinv,
                W, b.reshape(1, D), gamma.reshape(1, D), beta.reshape(1, D))
